# Initial kernel scaffold; baseline (speedup 1.0000x reference)
import functools

import jax
import jax.numpy as jnp
from jax import lax
from jax.experimental import pallas as pl
from jax.experimental.pallas import tpu as pltpu
from jax.experimental.pallas import tpu_sc as plsc

N = 10000
E = 160000
DIM = 256
H = 128
NP = 10240
SLICE = 640
EPT = 10032
EPAD = EPT * 16
EBUF = EPAD + 128
CHUNK = 80
FCH = 8

_SC_PARAMS = pltpu.CompilerParams(needs_layout_passes=False)
_mesh = plsc.VectorSubcoreMesh(core_axis_name="c", subcore_axis_name="s")


def _mm_body(x_ref, w_ref, ws_ref, h_ref, hs_ref):
    xb = x_ref[...]
    dn = (((1,), (1,)), ((), ()))
    h_ref[0] = lax.dot_general(xb, w_ref[0], dn, preferred_element_type=jnp.float32)
    hs_ref[0] = lax.dot_general(xb, ws_ref[0], dn, preferred_element_type=jnp.float32)


def _tc_project(x, w2, ws2):
    nb = 20
    rb = N // nb
    return pl.pallas_call(
        _mm_body,
        grid=(2, nb),
        in_specs=[
            pl.BlockSpec((rb, DIM), lambda j, i: (i, 0)),
            pl.BlockSpec((1, H, DIM), lambda j, i: (j, 0, 0)),
            pl.BlockSpec((1, H, DIM), lambda j, i: (j, 0, 0)),
        ],
        out_specs=[
            pl.BlockSpec((1, rb, H), lambda j, i: (j, i, 0)),
            pl.BlockSpec((1, rb, H), lambda j, i: (j, i, 0)),
        ],
        out_shape=[
            jax.ShapeDtypeStruct((2, N, H), jnp.float32),
            jax.ShapeDtypeStruct((2, N, H), jnp.float32),
        ],
    )(x, w2, ws2)


def _sigmoid16(v):
    return 1.0 / (1.0 + jnp.exp(-v))


@functools.partial(
    pl.kernel,
    mesh=_mesh,
    compiler_params=_SC_PARAMS,
    out_type=jax.ShapeDtypeStruct((2, N, H), jnp.float32),
    scratch_types=[
        pltpu.VMEM((16,), jnp.int32),
        pltpu.VMEM((16,), jnp.int32),
        pltpu.VMEM((SLICE,), jnp.float32),
        pltpu.VMEM((SLICE,), jnp.float32),
        pltpu.VMEM((16,), jnp.float32),
        pltpu.VMEM((16,), jnp.float32),
        pltpu.VMEM_SHARED((16, NP), jnp.float32),
        pltpu.VMEM_SHARED((16, NP), jnp.float32),
        pltpu.VMEM_SHARED((NP,), jnp.float32),
        pltpu.VMEM_SHARED((EBUF,), jnp.float32),
    ],
)
def _sc_message(row_hbm, gidx0_hbm, gidx1_hbm, sw_hbm, rep_a_hbm, rep_b_hbm,
                ns_hbm, reps_hbm, st_hbm, en_hbm, h2_hbm, hs2_hbm, out2,
                sbuf, ebuf, deg_sl, reps_sl, iva, ivb,
                stg_sn, stg_dg, invsn_sp, coeff_sp):
    c = lax.axis_index("c")
    s = lax.axis_index("s")
    cN = c * N
    i16 = lax.iota(jnp.int32, 16)

    def pass1(ra, rb, nsb, snl, dgl, rsl, csl, swl, cof):
        pltpu.sync_copy(rep_a_hbm, ra)
        pltpu.sync_copy(rep_b_hbm, rb)
        pltpu.sync_copy(ns_hbm, nsb)
        e0 = s * EPT
        pltpu.sync_copy(row_hbm.at[pl.ds(e0, EPT)], rsl)
        pltpu.sync_copy(gidx0_hbm.at[pl.ds(e0, EPT)], csl)
        pltpu.sync_copy(sw_hbm.at[pl.ds(e0, EPT)], swl)

        zf = jnp.zeros((16,), jnp.float32)

        def zero_np(u, _):
            snl[pl.ds(u * 16, 16)] = zf
            dgl[pl.ds(u * 16, 16)] = zf
            return 0

        lax.fori_loop(0, NP // 16, zero_np, 0)

        def edge_group(u, _):
            off = u * 16
            r16 = rsl[pl.ds(off, 16)]
            c16 = csl[pl.ds(off, 16)]
            sw16 = swl[pl.ds(off, 16)]
            ga = plsc.load_gather(ra, [r16])
            gb = plsc.load_gather(rb, [c16])
            nn = plsc.load_gather(nsb, [c16])
            gate = _sigmoid16(ga + gb)
            tn = 1.0 - 2.0 / (jnp.exp(2.0 * nn) + 1.0)
            cof[pl.ds(off, 16)] = sw16 * gate * tn
            plsc.addupdate_scatter(snl, [r16], sw16)
            plsc.addupdate_scatter(dgl, [r16], gate)
            return 0

        lax.fori_loop(0, EPT // 16, edge_group, 0)

        pltpu.sync_copy(snl, stg_sn.at[s])
        pltpu.sync_copy(dgl, stg_dg.at[s])
        pltpu.sync_copy(cof, coeff_sp.at[pl.ds(e0, EPT)])
        plsc.subcore_barrier()

        def combine(tmp, invl):
            nb = s * SLICE
            for p in range(16):
                pltpu.sync_copy(stg_sn.at[p, pl.ds(nb, SLICE)], tmp.at[p])

            def red_sn(g, _):
                o = g * 16
                acc = tmp[0, pl.ds(o, 16)]
                for p in range(1, 16):
                    acc = acc + tmp[p, pl.ds(o, 16)]
                invl[pl.ds(o, 16)] = 1.0 / (acc + 1e-6)
                return 0

            lax.fori_loop(0, SLICE // 16, red_sn, 0)
            pltpu.sync_copy(invl, invsn_sp.at[pl.ds(nb, SLICE)])

            for p in range(16):
                pltpu.sync_copy(stg_dg.at[p, pl.ds(nb, SLICE)], tmp.at[p])

            def red_dg(g, _):
                o = g * 16
                acc = tmp[0, pl.ds(o, 16)]
                for p in range(1, 16):
                    acc = acc + tmp[p, pl.ds(o, 16)]
                deg_sl[pl.ds(o, 16)] = acc
                return 0

            lax.fori_loop(0, SLICE // 16, red_dg, 0)

        pl.run_scoped(
            combine,
            pltpu.VMEM((16, SLICE), jnp.float32),
            pltpu.VMEM((SLICE,), jnp.float32),
        )
        plsc.subcore_barrier()

    pl.run_scoped(
        pass1,
        pltpu.VMEM((NP,), jnp.float32),
        pltpu.VMEM((NP,), jnp.float32),
        pltpu.VMEM((NP,), jnp.float32),
        pltpu.VMEM((NP,), jnp.float32),
        pltpu.VMEM((NP,), jnp.float32),
        pltpu.VMEM((EPT,), jnp.int32),
        pltpu.VMEM((EPT,), jnp.int32),
        pltpu.VMEM((EPT,), jnp.float32),
        pltpu.VMEM((EPT,), jnp.float32),
    )

    def pass2(acc, invsn, gbuf, rbuf, gxbuf, cbuf, hbuf):
        pltpu.sync_copy(st_hbm, sbuf)
        pltpu.sync_copy(en_hbm, ebuf)
        pltpu.sync_copy(invsn_sp, invsn)
        pltpu.sync_copy(reps_hbm.at[pl.ds(s * SLICE, SLICE)], reps_sl)

        start = jnp.sum(jnp.where(i16 == s, sbuf[...], 0))
        end = jnp.sum(jnp.where(i16 == s, ebuf[...], 0))
        nch = (end - start + (CHUNK - 1)) // CHUNK

        zf = jnp.zeros((16,), jnp.float32)

        def zero_acc(u, _):
            acc[pl.ds(u * 16, 16)] = zf
            return 0

        lax.fori_loop(0, SLICE * H // 16, zero_acc, 0)

        nb = s * SLICE

        def chunk(i, _):
            base = start + i * CHUNK
            pltpu.sync_copy(row_hbm.at[pl.ds(base, CHUNK)], rbuf)

            @pl.when(c == 0)
            def _():
                pltpu.sync_copy(gidx0_hbm.at[pl.ds(base, CHUNK)], gxbuf)

            @pl.when(c == 1)
            def _():
                pltpu.sync_copy(gidx1_hbm.at[pl.ds(base, CHUNK)], gxbuf)

            pltpu.sync_copy(coeff_sp.at[pl.ds(base, CHUNK)], cbuf)
            pltpu.sync_copy(h2_hbm.at[gxbuf], gbuf)

            for g in range(CHUNK // 16):
                eo = g * 16
                r16 = rbuf[pl.ds(eo, 16)]
                co16 = cbuf[pl.ds(eo, 16)] * plsc.load_gather(invsn, [r16])
                eidx = base + eo + i16
                co16 = jnp.where(eidx < end, co16, 0.0)
                lr = jnp.minimum(r16 - nb, SLICE - 1) * H
                e16 = i16 + eo

                def colm(m, _):
                    v = plsc.load_gather(gbuf, [e16, jnp.full((16,), m, jnp.int32)])
                    plsc.addupdate_scatter(acc, [lr + m], v * co16)
                    return 0

                lax.fori_loop(0, H, colm, 0)
            return 0

        lax.fori_loop(0, nch, chunk, 0)

        def final_chunk(ch, _):
            base = nb + ch * CHUNK

            @pl.when(base < N)
            def _():
                pltpu.sync_copy(hs2_hbm.at[pl.ds(cN + base, CHUNK)], hbuf)
                for g in range(CHUNK // 16):
                    lo = ch * CHUNK + g * 16
                    d16 = deg_sl[pl.ds(lo, 16)]
                    iva[...] = 1.0 / (d16 + 1e-6)
                    ivb[...] = _sigmoid16(reps_sl[pl.ds(lo, 16)])

                    def frow(k, _):
                        kk = jnp.full((16,), k, jnp.int32)
                        dsp = plsc.load_gather(iva, [kk])
                        gsp = plsc.load_gather(ivb, [kk])
                        gr = jnp.full((16,), g * 16 + k, jnp.int32)
                        ab = (lo + k) * H
                        for m in range(H // 16):
                            ci = i16 + m * 16
                            a = plsc.load_gather(acc, [ab + ci])
                            hv = plsc.load_gather(hbuf, [gr, ci])
                            o = a * dsp + gsp * hv
                            o = jnp.where(o >= 0.0, o, o * 0.01)
                            plsc.store_scatter(gbuf, [gr, ci], o)
                        return 0

                    lax.fori_loop(0, 16, frow, 0)
                pltpu.sync_copy(gbuf, out2.at[c, pl.ds(base, CHUNK)])

            return 0

        lax.fori_loop(0, FCH, final_chunk, 0)

    pl.run_scoped(
        pass2,
        pltpu.VMEM((SLICE * H,), jnp.float32),
        pltpu.VMEM((NP,), jnp.float32),
        pltpu.VMEM((CHUNK, H), jnp.float32),
        pltpu.VMEM((CHUNK,), jnp.int32),
        pltpu.VMEM((CHUNK,), jnp.int32),
        pltpu.VMEM((CHUNK,), jnp.float32),
        pltpu.VMEM((CHUNK, H), jnp.float32),
    )


def kernel(x, edge_index, sim_weight, rep, node_signal, W, W_self,
           alpha, beta, alpha_self, temp):
    row = edge_index[0].astype(jnp.int32)
    col = edge_index[1].astype(jnp.int32)
    sw = sim_weight.astype(jnp.float32)

    order = jnp.argsort(row)
    npad = EBUF - E
    row_p = jnp.concatenate([row[order], jnp.full((npad,), N - 1, jnp.int32)])
    col_p = jnp.concatenate([col[order], jnp.zeros((npad,), jnp.int32)])
    sw_p = jnp.concatenate([sw[order], jnp.zeros((npad,), jnp.float32)])
    starts = jnp.searchsorted(
        row_p[:EPAD], jnp.arange(16, dtype=jnp.int32) * SLICE).astype(jnp.int32)
    ends = jnp.concatenate([starts[1:], jnp.array([EPAD], jnp.int32)])

    pad_n = NP - N
    zn = jnp.zeros((pad_n,), jnp.float32)
    rep_a = jnp.concatenate([(alpha / temp) * rep, zn])
    rep_b = jnp.concatenate([(beta / temp) * rep, zn])
    ns_p = jnp.concatenate([node_signal, zn])
    reps = jnp.concatenate([(alpha_self / temp) * rep, zn])

    w2 = W.reshape(2, H, DIM)
    ws2 = W_self.reshape(2, H, DIM)
    h2, hs2 = _tc_project(x, w2, ws2)
    h2f = h2.reshape(2 * N, H)
    hs2f = hs2.reshape(2 * N, H)

    out2 = _sc_message(row_p, col_p, col_p + N, sw_p, rep_a, rep_b, ns_p,
                       reps, starts, ends, h2f, hs2f)
    return jnp.concatenate([out2[0], out2[1]], axis=1)


# speedup vs baseline: 2.0986x; 2.0986x over previous
import functools

import jax
import jax.numpy as jnp
from jax import lax
from jax.experimental import pallas as pl
from jax.experimental.pallas import tpu as pltpu
from jax.experimental.pallas import tpu_sc as plsc

N = 10000
E = 160000
DIM = 256
H = 128
NP = 10240
SLICE = 640
EPT = 10032
EPAD = EPT * 16
EBUF = EPAD + 128
CHUNK = 80
P1C = 3344
FCH = 8

_SC_PARAMS = pltpu.CompilerParams(needs_layout_passes=False)
_mesh = plsc.VectorSubcoreMesh(core_axis_name="c", subcore_axis_name="s")


def _mm_body(x_ref, w_ref, ws_ref, h_ref, hs_ref):
    xb = x_ref[...]
    dn = (((1,), (1,)), ((), ()))
    h_ref[0] = lax.dot_general(xb, w_ref[0], dn, preferred_element_type=jnp.float32)
    hs_ref[0] = lax.dot_general(xb, ws_ref[0], dn, preferred_element_type=jnp.float32)


def _tc_project(x, w2, ws2):
    nb = 10
    rb = NP // nb
    return pl.pallas_call(
        _mm_body,
        grid=(2, nb),
        in_specs=[
            pl.BlockSpec((rb, DIM), lambda j, i: (i, 0)),
            pl.BlockSpec((1, H, DIM), lambda j, i: (j, 0, 0)),
            pl.BlockSpec((1, H, DIM), lambda j, i: (j, 0, 0)),
        ],
        out_specs=[
            pl.BlockSpec((1, rb, H), lambda j, i: (j, i, 0)),
            pl.BlockSpec((1, rb, H), lambda j, i: (j, i, 0)),
        ],
        out_shape=[
            jax.ShapeDtypeStruct((2, NP, H), jnp.float32),
            jax.ShapeDtypeStruct((2, NP, H), jnp.float32),
        ],
    )(x, w2, ws2)


def _sigmoid16(v):
    return 1.0 / (1.0 + jnp.exp(-v))


@functools.partial(
    pl.kernel,
    mesh=_mesh,
    compiler_params=_SC_PARAMS,
    out_type=jax.ShapeDtypeStruct((2, NP, H), jnp.float32),
    scratch_types=[
        pltpu.VMEM((16,), jnp.int32),
        pltpu.VMEM((16,), jnp.int32),
        pltpu.VMEM((SLICE,), jnp.float32),
        pltpu.VMEM((SLICE,), jnp.float32),
        pltpu.VMEM((SLICE,), jnp.float32),
        pltpu.VMEM((16,), jnp.float32),
        pltpu.VMEM((16,), jnp.float32),
        pltpu.VMEM_SHARED((16, NP), jnp.float32),
        pltpu.VMEM_SHARED((16, NP), jnp.float32),
        pltpu.VMEM_SHARED((EBUF,), jnp.float32),
    ],
)
def _sc_message(row_hbm, gidx0_hbm, sw_hbm, rep_a_hbm, rep_b_hbm,
                ns_hbm, reps_hbm, st_hbm, en_hbm, h2_hbm, hs2_hbm, out2,
                sbuf, ebuf, deg_sl, inv_sl, reps_sl, iva, ivb,
                stg_sn, stg_dg, coeff_sp):
    c = lax.axis_index("c")
    s = lax.axis_index("s")
    cNP = c * NP
    i16 = lax.iota(jnp.int32, 16)

    def pass1(ra, rb, nsb, snl, dgl, rsl, csl, swl, cof):
        pltpu.sync_copy(rep_a_hbm, ra)
        pltpu.sync_copy(rep_b_hbm, rb)
        pltpu.sync_copy(ns_hbm, nsb)

        zf = jnp.zeros((16,), jnp.float32)

        def zero_np(u, _):
            snl[pl.ds(u * 16, 16)] = zf
            dgl[pl.ds(u * 16, 16)] = zf
            return 0

        lax.fori_loop(0, NP // 16, zero_np, 0)

        for ech in range(EPT // P1C):
            e0 = s * EPT + ech * P1C
            pltpu.sync_copy(row_hbm.at[pl.ds(e0, P1C)], rsl)
            pltpu.sync_copy(gidx0_hbm.at[pl.ds(e0, P1C)], csl)
            pltpu.sync_copy(sw_hbm.at[pl.ds(e0, P1C)], swl)

            def edge_group(u, _):
                off = u * 16
                r16 = rsl[pl.ds(off, 16)]
                c16 = csl[pl.ds(off, 16)]
                sw16 = swl[pl.ds(off, 16)]
                ga = plsc.load_gather(ra, [r16])
                gb = plsc.load_gather(rb, [c16])
                nn = plsc.load_gather(nsb, [c16])
                gate = _sigmoid16(ga + gb)
                tn = 1.0 - 2.0 / (jnp.exp(2.0 * nn) + 1.0)
                cof[pl.ds(off, 16)] = sw16 * gate * tn
                plsc.addupdate_scatter(snl, [r16], sw16)
                plsc.addupdate_scatter(dgl, [r16], gate)
                return 0

            lax.fori_loop(0, P1C // 16, edge_group, 0)
            pltpu.sync_copy(cof, coeff_sp.at[pl.ds(e0, P1C)])

        pltpu.sync_copy(snl, stg_sn.at[s])
        pltpu.sync_copy(dgl, stg_dg.at[s])
        plsc.subcore_barrier()

        def combine(tmp):
            nb = s * SLICE
            for p in range(16):
                pltpu.sync_copy(stg_sn.at[p, pl.ds(nb, SLICE)], tmp.at[p])

            def red_sn(g, _):
                o = g * 16
                acc = tmp[0, pl.ds(o, 16)]
                for p in range(1, 16):
                    acc = acc + tmp[p, pl.ds(o, 16)]
                inv_sl[pl.ds(o, 16)] = 1.0 / (acc + 1e-6)
                return 0

            lax.fori_loop(0, SLICE // 16, red_sn, 0)

            for p in range(16):
                pltpu.sync_copy(stg_dg.at[p, pl.ds(nb, SLICE)], tmp.at[p])

            def red_dg(g, _):
                o = g * 16
                acc = tmp[0, pl.ds(o, 16)]
                for p in range(1, 16):
                    acc = acc + tmp[p, pl.ds(o, 16)]
                deg_sl[pl.ds(o, 16)] = acc
                return 0

            lax.fori_loop(0, SLICE // 16, red_dg, 0)

        pl.run_scoped(
            combine,
            pltpu.VMEM((16, SLICE), jnp.float32),
        )

    pl.run_scoped(
        pass1,
        pltpu.VMEM((NP,), jnp.float32),
        pltpu.VMEM((NP,), jnp.float32),
        pltpu.VMEM((NP,), jnp.float32),
        pltpu.VMEM((NP,), jnp.float32),
        pltpu.VMEM((NP,), jnp.float32),
        pltpu.VMEM((P1C,), jnp.int32),
        pltpu.VMEM((P1C,), jnp.int32),
        pltpu.VMEM((P1C,), jnp.float32),
        pltpu.VMEM((P1C,), jnp.float32),
    )

    def pass2(acc, gbuf, rbuf, gxbuf, cbuf):
        pltpu.sync_copy(st_hbm, sbuf)
        pltpu.sync_copy(en_hbm, ebuf)
        pltpu.sync_copy(reps_hbm.at[pl.ds(s * SLICE, SLICE)], reps_sl)

        start = jnp.sum(jnp.where(i16 == s, sbuf[...], 0))
        end = jnp.sum(jnp.where(i16 == s, ebuf[...], 0))
        abase = (start // 8) * 8
        nch = (end - abase + (CHUNK - 1)) // CHUNK

        zf = jnp.zeros((16,), jnp.float32)

        def zero_acc(u, _):
            acc[pl.ds(u * 16, 16)] = zf
            return 0

        lax.fori_loop(0, SLICE * H // 16, zero_acc, 0)

        nb = s * SLICE

        def chunk(i, _):
            base = abase + i * CHUNK
            pltpu.sync_copy(row_hbm.at[pl.ds(base, CHUNK)], rbuf)
            pltpu.sync_copy(gidx0_hbm.at[pl.ds(base, CHUNK)], gxbuf)
            pltpu.sync_copy(coeff_sp.at[pl.ds(base, CHUNK)], cbuf)
            for g in range(CHUNK // 16):
                go = g * 16
                gxbuf[pl.ds(go, 16)] = gxbuf[pl.ds(go, 16)] + cNP
            pltpu.sync_copy(h2_hbm.at[gxbuf], gbuf)

            for g in range(CHUNK // 16):
                eo = g * 16
                r16 = rbuf[pl.ds(eo, 16)]
                lr16 = jnp.minimum(jnp.maximum(r16 - nb, 0), SLICE - 1)
                co16 = cbuf[pl.ds(eo, 16)] * plsc.load_gather(inv_sl, [lr16])
                eidx = base + eo + i16
                ok = (eidx >= start) & (eidx < end)
                co16 = jnp.where(ok, co16, 0.0)
                lr = lr16 * H
                e16 = i16 + eo

                def colm(m, _):
                    v = plsc.load_gather(gbuf, [e16, jnp.full((16,), m, jnp.int32)])
                    plsc.addupdate_scatter(acc, [lr + m], v * co16)
                    return 0

                lax.fori_loop(0, H, colm, 0)
            return 0

        lax.fori_loop(0, nch, chunk, 0)

        def final_chunk(ch, _):
            lb = ch * 32
            pltpu.sync_copy(hs2_hbm.at[pl.ds(cNP + nb + lb, 32)],
                            gbuf.at[pl.ds(32, 32)])
            for g in range(2):
                lo = lb + g * 16
                d16 = deg_sl[pl.ds(lo, 16)]
                iva[...] = 1.0 / (d16 + 1e-6)
                ivb[...] = _sigmoid16(reps_sl[pl.ds(lo, 16)])

                def frow(k, _):
                    kk = jnp.full((16,), k, jnp.int32)
                    dsp = plsc.load_gather(iva, [kk])
                    gsp = plsc.load_gather(ivb, [kk])
                    gro = jnp.full((16,), g * 16 + k, jnp.int32)
                    ab = (lo + k) * H
                    for m in range(H // 16):
                        ci = i16 + m * 16
                        a = plsc.load_gather(acc, [ab + ci])
                        hv = plsc.load_gather(gbuf, [gro + 32, ci])
                        o = a * dsp + gsp * hv
                        o = jnp.where(o >= 0.0, o, o * 0.01)
                        plsc.store_scatter(gbuf, [gro, ci], o)
                    return 0

                lax.fori_loop(0, 16, frow, 0)
            pltpu.sync_copy(gbuf.at[pl.ds(0, 32)], out2.at[c, pl.ds(nb + lb, 32)])
            return 0

        lax.fori_loop(0, SLICE // 32, final_chunk, 0)

    pl.run_scoped(
        pass2,
        pltpu.VMEM((SLICE * H,), jnp.float32),
        pltpu.VMEM((CHUNK, H), jnp.float32),
        pltpu.VMEM((CHUNK,), jnp.int32),
        pltpu.VMEM((CHUNK,), jnp.int32),
        pltpu.VMEM((CHUNK,), jnp.float32),
    )


def kernel(x, edge_index, sim_weight, rep, node_signal, W, W_self,
           alpha, beta, alpha_self, temp):
    row = edge_index[0].astype(jnp.int32)
    col = edge_index[1].astype(jnp.int32)
    sw = sim_weight.astype(jnp.float32)

    order = jnp.argsort(row)
    npad = EBUF - E
    row_p = jnp.concatenate([row[order], jnp.full((npad,), NP - 1, jnp.int32)])
    col_p = jnp.concatenate([col[order], jnp.zeros((npad,), jnp.int32)])
    sw_p = jnp.concatenate([sw[order], jnp.zeros((npad,), jnp.float32)])
    starts = jnp.searchsorted(
        row_p[:EPAD], jnp.arange(16, dtype=jnp.int32) * SLICE).astype(jnp.int32)
    ends = jnp.concatenate([starts[1:], jnp.array([EPAD], jnp.int32)])

    pad_n = NP - N
    zn = jnp.zeros((pad_n,), jnp.float32)
    rep_a = jnp.concatenate([(alpha / temp) * rep, zn])
    rep_b = jnp.concatenate([(beta / temp) * rep, zn])
    ns_p = jnp.concatenate([node_signal, zn])
    reps = jnp.concatenate([(alpha_self / temp) * rep, zn])

    w2 = W.reshape(2, H, DIM)
    ws2 = W_self.reshape(2, H, DIM)
    x_p = jnp.concatenate([x, jnp.zeros((pad_n, DIM), jnp.float32)])
    h2, hs2 = _tc_project(x_p, w2, ws2)
    h2f = h2.reshape(2 * NP, H)
    hs2f = hs2.reshape(2 * NP, H)

    out2 = _sc_message(row_p, col_p, sw_p, rep_a, rep_b, ns_p,
                       reps, starts, ends, h2f, hs2f)
    return jnp.concatenate([out2[0, :N], out2[1, :N]], axis=1)
